# baseline (speedup 1.0000x reference)
import jax
import jax.numpy as jnp
from jax.experimental import pallas as pl
from jax.experimental.pallas import tpu as pltpu

_H = 128
_W = 2 * _H
_ROWS = 4096
_CORES = 1


def _mlp_body(x_ref, w1a_ref, w1b_ref, b1_ref, w2_ref, b2_ref, w3_ref, b3_ref,
              r_ref):
    x = x_ref[...]
    rr = x.shape[0] // 128

    def net(w1):
        h = jnp.dot(x, w1, preferred_element_type=jnp.float32)
        h = jnp.maximum(h + b1_ref[...], 0.0)
        h = jnp.dot(h, w2_ref[...], preferred_element_type=jnp.float32)
        h = jnp.maximum(h + b2_ref[...], 0.0)
        o = jnp.dot(h, w3_ref[...], preferred_element_type=jnp.float32) + b3_ref[...]
        d = o[:, :_H] - o[:, _H:]
        sq = (d * d).reshape(rr, 128, 128)
        return jnp.sum(sq, axis=-1) * (1.0 / _H)

    ra = net(w1a_ref[...])
    rb = net(w1b_ref[...])
    r1 = jnp.concatenate([ra[:, :64], rb[:, :64]], axis=1)
    r2 = jnp.concatenate([ra[:, 64:], rb[:, 64:]], axis=1)
    rtmp = jnp.stack([r1, r2], axis=1).reshape(2 * rr, 128)
    col = jax.lax.broadcasted_iota(jnp.int32, rtmp.shape, 1)
    src = jnp.where(col % 2 == 0, col // 2, 64 + col // 2)
    r_ref[...] = jnp.take_along_axis(rtmp, src, axis=1)


def _norm_body(mean_ref, m2_ref, count_ref, r_ref, out_ref):
    r = r_ref[...]
    n = jnp.float32(r.size)
    bm = jnp.sum(r) / n
    dv = r - bm
    bm2 = jnp.sum(dv * dv)
    cnt = count_ref[0]
    new_count = cnt + n
    delta = bm - mean_ref[0]
    new_mean = mean_ref[0] + delta * n / new_count
    new_m2 = m2_ref[0] + bm2 + delta * delta * cnt * n / new_count
    std = jnp.where(new_count > 1.0, jnp.sqrt(new_m2 / (new_count - 1.0)), 1.0)
    out_ref[...] = (r - new_mean) * (1.0 / (std + 1e-8))


def kernel(obs, reward_mean, reward_m2, reward_count,
           tW1, tb1, tW2, tb2, tW3, tb3,
           pW1, pb1, pW2, pb2, pW3, pb3):
    batch, obs_dim = obs.shape
    z64 = jnp.zeros((obs_dim, _W), jnp.float32)
    z = jnp.zeros((_H, _H), jnp.float32)
    w1 = jnp.concatenate([tW1.T, pW1.T], axis=1)
    w1a = jnp.concatenate([w1, z64], axis=0)
    w1b = jnp.concatenate([z64, w1], axis=0)
    b1 = jnp.concatenate([tb1, pb1])[None, :]
    w2 = jnp.concatenate(
        [jnp.concatenate([tW2.T, z], axis=1),
         jnp.concatenate([z, pW2.T], axis=1)], axis=0)
    b2 = jnp.concatenate([tb2, pb2])[None, :]
    w3 = jnp.concatenate(
        [jnp.concatenate([tW3.T, z], axis=1),
         jnp.concatenate([z, pW3.T], axis=1)], axis=0)
    b3 = jnp.concatenate([tb3, pb3])[None, :]

    x2 = obs.reshape(batch // 2, 128)
    rows = _ROWS // 2
    steps = batch // (_CORES * _ROWS)
    rrows = _ROWS // 128

    rewards = pl.pallas_call(
        _mlp_body,
        grid=(_CORES, steps),
        in_specs=[
            pl.BlockSpec((rows, 128), lambda c, i, s=steps: (c * s + i, 0)),
            pl.BlockSpec((_H, _W), lambda c, i: (0, 0)),
            pl.BlockSpec((_H, _W), lambda c, i: (0, 0)),
            pl.BlockSpec((1, _W), lambda c, i: (0, 0)),
            pl.BlockSpec((_W, _W), lambda c, i: (0, 0)),
            pl.BlockSpec((1, _W), lambda c, i: (0, 0)),
            pl.BlockSpec((_W, _W), lambda c, i: (0, 0)),
            pl.BlockSpec((1, _W), lambda c, i: (0, 0)),
        ],
        out_specs=pl.BlockSpec((rrows, 128), lambda c, i, s=steps: (c * s + i, 0)),
        out_shape=jax.ShapeDtypeStruct((batch // 128, 128), jnp.float32),
        compiler_params=pltpu.CompilerParams(
            dimension_semantics=("parallel", "arbitrary"),
        ),
    )(x2, w1a, w1b, b1, w2, b2, w3, b3)

    normalized = pl.pallas_call(
        _norm_body,
        in_specs=[
            pl.BlockSpec(memory_space=pltpu.SMEM),
            pl.BlockSpec(memory_space=pltpu.SMEM),
            pl.BlockSpec(memory_space=pltpu.SMEM),
            pl.BlockSpec((batch // 128, 128), lambda: (0, 0)),
        ],
        out_specs=pl.BlockSpec((batch // 128, 128), lambda: (0, 0)),
        out_shape=jax.ShapeDtypeStruct((batch // 128, 128), jnp.float32),
    )(reward_mean, reward_m2, reward_count, rewards)

    return normalized.reshape(batch)


# speedup vs baseline: 1.4913x; 1.4913x over previous
import jax
import jax.numpy as jnp
from jax.experimental import pallas as pl
from jax.experimental.pallas import tpu as pltpu

_H = 128
_W = 2 * _H
_NB = 4096


def _mlp_body(x_ref, w1_ref, b1_ref, w2_ref, b2_ref, w3_ref, b3_ref, r_ref):
    x = x_ref[...]
    nb = x.shape[1]
    reps = nb // 128
    b1 = pltpu.repeat(b1_ref[...], reps, axis=1)
    b2 = pltpu.repeat(b2_ref[...], reps, axis=1)
    b3 = pltpu.repeat(b3_ref[...], reps, axis=1)
    h = jnp.dot(w1_ref[...], x, preferred_element_type=jnp.float32)
    h = jnp.maximum(h + b1, 0.0)
    h = jnp.dot(w2_ref[...], h, preferred_element_type=jnp.float32)
    h = jnp.maximum(h + b2, 0.0)
    o = jnp.dot(w3_ref[...], h, preferred_element_type=jnp.float32) + b3
    d = o[:_H, :] - o[_H:, :]
    sq = (d * d).T
    r3 = sq.reshape(reps, 128, 128)
    r_ref[...] = jnp.sum(r3, axis=-1) * (1.0 / _H)


def _norm_body(mean_ref, m2_ref, count_ref, r_ref, out_ref):
    r = r_ref[...]
    n = jnp.float32(r.size)
    bm = jnp.sum(r) / n
    dv = r - bm
    bm2 = jnp.sum(dv * dv)
    cnt = count_ref[0]
    new_count = cnt + n
    delta = bm - mean_ref[0]
    new_mean = mean_ref[0] + delta * n / new_count
    new_m2 = m2_ref[0] + bm2 + delta * delta * cnt * n / new_count
    std = jnp.where(new_count > 1.0, jnp.sqrt(new_m2 / (new_count - 1.0)), 1.0)
    out_ref[...] = (r - new_mean) * (1.0 / (std + 1e-8))


def kernel(obs, reward_mean, reward_m2, reward_count,
           tW1, tb1, tW2, tb2, tW3, tb3,
           pW1, pb1, pW2, pb2, pW3, pb3):
    batch, obs_dim = obs.shape
    z = jnp.zeros((_H, _H), jnp.float32)
    w1 = jnp.concatenate([tW1, pW1], axis=0)
    w2 = jnp.concatenate(
        [jnp.concatenate([tW2, z], axis=1),
         jnp.concatenate([z, pW2], axis=1)], axis=0)
    w3 = jnp.concatenate(
        [jnp.concatenate([tW3, z], axis=1),
         jnp.concatenate([z, pW3], axis=1)], axis=0)
    b1 = jnp.broadcast_to(jnp.concatenate([tb1, pb1])[:, None], (_W, 128))
    b2 = jnp.broadcast_to(jnp.concatenate([tb2, pb2])[:, None], (_W, 128))
    b3 = jnp.broadcast_to(jnp.concatenate([tb3, pb3])[:, None], (_W, 128))

    xt = obs.T
    steps = batch // _NB
    rrows = _NB // 128

    rewards = pl.pallas_call(
        _mlp_body,
        grid=(steps,),
        in_specs=[
            pl.BlockSpec((obs_dim, _NB), lambda i: (0, i)),
            pl.BlockSpec((_W, obs_dim), lambda i: (0, 0)),
            pl.BlockSpec((_W, 128), lambda i: (0, 0)),
            pl.BlockSpec((_W, _W), lambda i: (0, 0)),
            pl.BlockSpec((_W, 128), lambda i: (0, 0)),
            pl.BlockSpec((_W, _W), lambda i: (0, 0)),
            pl.BlockSpec((_W, 128), lambda i: (0, 0)),
        ],
        out_specs=pl.BlockSpec((rrows, 128), lambda i: (i, 0)),
        out_shape=jax.ShapeDtypeStruct((batch // 128, 128), jnp.float32),
        compiler_params=pltpu.CompilerParams(
            dimension_semantics=("arbitrary",),
        ),
    )(xt, w1, b1, w2, b2, w3, b3)

    normalized = pl.pallas_call(
        _norm_body,
        in_specs=[
            pl.BlockSpec(memory_space=pltpu.SMEM),
            pl.BlockSpec(memory_space=pltpu.SMEM),
            pl.BlockSpec(memory_space=pltpu.SMEM),
            pl.BlockSpec((batch // 128, 128), lambda: (0, 0)),
        ],
        out_specs=pl.BlockSpec((batch // 128, 128), lambda: (0, 0)),
        out_shape=jax.ShapeDtypeStruct((batch // 128, 128), jnp.float32),
    )(reward_mean, reward_m2, reward_count, rewards)

    return normalized.reshape(batch)


# speedup vs baseline: 2.2850x; 1.5323x over previous
import jax
import jax.numpy as jnp
from jax.experimental import pallas as pl
from jax.experimental.pallas import tpu as pltpu

_H = 128
_W = 2 * _H
_NB = 8192


def _mlp_body(x_ref, w1_ref, b1_ref, w2_ref, b2_ref, w3_ref, b3_ref, r_ref):
    x = x_ref[...]
    nb = x.shape[1]
    reps = nb // 128
    b1 = pltpu.repeat(b1_ref[...], reps, axis=1)
    b2 = pltpu.repeat(b2_ref[...], reps, axis=1)
    b3 = pltpu.repeat(b3_ref[...], reps, axis=1)
    h = jnp.dot(w1_ref[...], x, preferred_element_type=jnp.float32)
    h = jnp.maximum(h + b1, 0.0)
    h = jnp.dot(w2_ref[...], h, preferred_element_type=jnp.float32)
    h = jnp.maximum(h + b2, 0.0)
    o = jnp.dot(w3_ref[...], h, preferred_element_type=jnp.float32) + b3
    d = o[:_H, :] - o[_H:, :]
    s16 = (d * d).reshape(16, 8, nb)
    r_ref[...] = jnp.sum(s16, axis=0)[None]


def _norm_body(mean_ref, m2_ref, count_ref, r_ref, out_ref):
    r8 = r_ref[...]
    r = jnp.sum(r8, axis=1) * (1.0 / _H)
    n = jnp.float32(r.size)
    bm = jnp.sum(r) / n
    dv = r - bm
    bm2 = jnp.sum(dv * dv)
    cnt = count_ref[0]
    new_count = cnt + n
    delta = bm - mean_ref[0]
    new_mean = mean_ref[0] + delta * n / new_count
    new_m2 = m2_ref[0] + bm2 + delta * delta * cnt * n / new_count
    std = jnp.where(new_count > 1.0, jnp.sqrt(new_m2 / (new_count - 1.0)), 1.0)
    out_ref[...] = (r - new_mean) * (1.0 / (std + 1e-8))


def kernel(obs, reward_mean, reward_m2, reward_count,
           tW1, tb1, tW2, tb2, tW3, tb3,
           pW1, pb1, pW2, pb2, pW3, pb3):
    batch, obs_dim = obs.shape
    z = jnp.zeros((_H, _H), jnp.float32)
    w1 = jnp.concatenate([tW1, pW1], axis=0)
    w2 = jnp.concatenate(
        [jnp.concatenate([tW2, z], axis=1),
         jnp.concatenate([z, pW2], axis=1)], axis=0)
    w3 = jnp.concatenate(
        [jnp.concatenate([tW3, z], axis=1),
         jnp.concatenate([z, pW3], axis=1)], axis=0)
    b1 = jnp.broadcast_to(jnp.concatenate([tb1, pb1])[:, None], (_W, 128))
    b2 = jnp.broadcast_to(jnp.concatenate([tb2, pb2])[:, None], (_W, 128))
    b3 = jnp.broadcast_to(jnp.concatenate([tb3, pb3])[:, None], (_W, 128))

    xt = obs.T
    steps = batch // _NB

    partial = pl.pallas_call(
        _mlp_body,
        grid=(steps,),
        in_specs=[
            pl.BlockSpec((obs_dim, _NB), lambda i: (0, i)),
            pl.BlockSpec((_W, obs_dim), lambda i: (0, 0)),
            pl.BlockSpec((_W, 128), lambda i: (0, 0)),
            pl.BlockSpec((_W, _W), lambda i: (0, 0)),
            pl.BlockSpec((_W, 128), lambda i: (0, 0)),
            pl.BlockSpec((_W, _W), lambda i: (0, 0)),
            pl.BlockSpec((_W, 128), lambda i: (0, 0)),
        ],
        out_specs=pl.BlockSpec((1, 8, _NB), lambda i: (i, 0, 0)),
        out_shape=jax.ShapeDtypeStruct((steps, 8, _NB), jnp.float32),
        compiler_params=pltpu.CompilerParams(
            dimension_semantics=("arbitrary",),
        ),
    )(xt, w1, b1, w2, b2, w3, b3)

    normalized = pl.pallas_call(
        _norm_body,
        in_specs=[
            pl.BlockSpec(memory_space=pltpu.SMEM),
            pl.BlockSpec(memory_space=pltpu.SMEM),
            pl.BlockSpec(memory_space=pltpu.SMEM),
            pl.BlockSpec((steps, 8, _NB), lambda: (0, 0, 0)),
        ],
        out_specs=pl.BlockSpec((steps, _NB), lambda: (0, 0)),
        out_shape=jax.ShapeDtypeStruct((steps, _NB), jnp.float32),
    )(reward_mean, reward_m2, reward_count, partial)

    return normalized.reshape(batch)


# speedup vs baseline: 2.6381x; 1.1545x over previous
import jax
import jax.numpy as jnp
from jax.experimental import pallas as pl
from jax.experimental.pallas import tpu as pltpu

_H = 128
_W = 2 * _H
_NB = 8192


def _mlp_body(x_ref, w1_ref, b1_ref, w2_ref, b2_ref, w3_ref, b3_ref, r_ref):
    x = x_ref[...]
    nb = x.shape[1]
    reps = nb // 128
    b1 = pltpu.repeat(b1_ref[...], reps, axis=1)
    b2 = pltpu.repeat(b2_ref[...], reps, axis=1)
    b3 = pltpu.repeat(b3_ref[...], reps, axis=1)
    h = jnp.dot(w1_ref[...], x, preferred_element_type=jnp.float32)
    h = jnp.maximum(h + b1, 0.0)
    h = jnp.dot(w2_ref[...], h, preferred_element_type=jnp.float32)
    h = jnp.maximum(h + b2, 0.0)
    d = jnp.dot(w3_ref[...], h, preferred_element_type=jnp.float32) + b3
    s16 = (d * d).reshape(16, 8, nb)
    r_ref[...] = jnp.sum(s16, axis=0)[None]


def _norm_body(mean_ref, m2_ref, count_ref, r_ref, out_ref):
    r8 = r_ref[...]
    r = jnp.sum(r8, axis=1) * (1.0 / _H)
    n = jnp.float32(r.size)
    bm = jnp.sum(r) / n
    dv = r - bm
    bm2 = jnp.sum(dv * dv)
    cnt = count_ref[0]
    new_count = cnt + n
    delta = bm - mean_ref[0]
    new_mean = mean_ref[0] + delta * n / new_count
    new_m2 = m2_ref[0] + bm2 + delta * delta * cnt * n / new_count
    std = jnp.where(new_count > 1.0, jnp.sqrt(new_m2 / (new_count - 1.0)), 1.0)
    out_ref[...] = (r - new_mean) * (1.0 / (std + 1e-8))


def kernel(obs, reward_mean, reward_m2, reward_count,
           tW1, tb1, tW2, tb2, tW3, tb3,
           pW1, pb1, pW2, pb2, pW3, pb3):
    batch, obs_dim = obs.shape
    z = jnp.zeros((_H, _H), jnp.float32)
    w1 = jnp.concatenate([tW1, pW1], axis=0)
    w2 = jnp.concatenate(
        [jnp.concatenate([tW2, z], axis=1),
         jnp.concatenate([z, pW2], axis=1)], axis=0)
    w3 = jnp.concatenate([tW3, -pW3], axis=1)
    b1 = jnp.broadcast_to(jnp.concatenate([tb1, pb1])[:, None], (_W, 128))
    b2 = jnp.broadcast_to(jnp.concatenate([tb2, pb2])[:, None], (_W, 128))
    b3 = jnp.broadcast_to((tb3 - pb3)[:, None], (_H, 128))

    xt = obs.T
    steps = batch // _NB

    partial = pl.pallas_call(
        _mlp_body,
        grid=(steps,),
        in_specs=[
            pl.BlockSpec((obs_dim, _NB), lambda i: (0, i)),
            pl.BlockSpec((_W, obs_dim), lambda i: (0, 0)),
            pl.BlockSpec((_W, 128), lambda i: (0, 0)),
            pl.BlockSpec((_W, _W), lambda i: (0, 0)),
            pl.BlockSpec((_W, 128), lambda i: (0, 0)),
            pl.BlockSpec((_H, _W), lambda i: (0, 0)),
            pl.BlockSpec((_H, 128), lambda i: (0, 0)),
        ],
        out_specs=pl.BlockSpec((1, 8, _NB), lambda i: (i, 0, 0)),
        out_shape=jax.ShapeDtypeStruct((steps, 8, _NB), jnp.float32),
        compiler_params=pltpu.CompilerParams(
            dimension_semantics=("arbitrary",),
        ),
    )(xt, w1, b1, w2, b2, w3, b3)

    normalized = pl.pallas_call(
        _norm_body,
        in_specs=[
            pl.BlockSpec(memory_space=pltpu.SMEM),
            pl.BlockSpec(memory_space=pltpu.SMEM),
            pl.BlockSpec(memory_space=pltpu.SMEM),
            pl.BlockSpec((steps, 8, _NB), lambda: (0, 0, 0)),
        ],
        out_specs=pl.BlockSpec((steps, _NB), lambda: (0, 0)),
        out_shape=jax.ShapeDtypeStruct((steps, _NB), jnp.float32),
    )(reward_mean, reward_m2, reward_count, partial)

    return normalized.reshape(batch)


# speedup vs baseline: 2.8020x; 1.0621x over previous
import functools

import jax
import jax.numpy as jnp
from jax.experimental import pallas as pl
from jax.experimental.pallas import tpu as pltpu

_H = 128
_W = 2 * _H
_NB = 8192


def _body(steps, x_ref, w1_ref, b1_ref, w2_ref, b2_ref, w3_ref, b3_ref,
          mean_ref, m2_ref, count_ref,
          out_ref, racc_ref, acc1_ref, acc2_ref):
    i = pl.program_id(0)
    x = x_ref[...]
    reps = _NB // 128
    b1 = pltpu.repeat(b1_ref[...], reps, axis=1)
    b2 = pltpu.repeat(b2_ref[...], reps, axis=1)
    b3 = pltpu.repeat(b3_ref[...], reps, axis=1)
    h = jnp.dot(w1_ref[...], x, preferred_element_type=jnp.float32)
    h = jnp.maximum(h + b1, 0.0)
    h = jnp.dot(w2_ref[...], h, preferred_element_type=jnp.float32)
    h = jnp.maximum(h + b2, 0.0)
    d = jnp.dot(w3_ref[...], h, preferred_element_type=jnp.float32) + b3
    s16 = (d * d).reshape(16, 8, _NB)
    s8 = jnp.sum(s16, axis=0)
    r_row = jnp.sum(s8.reshape(1, 8, _NB), axis=1) * (1.0 / _H)
    racc_ref[pl.ds(i, 1)] = r_row[:, None, :]

    @pl.when(i == 0)
    def _init():
        acc1_ref[...] = jnp.zeros_like(acc1_ref)
        acc2_ref[...] = jnp.zeros_like(acc2_ref)

    acc1_ref[...] += r_row
    acc2_ref[...] += r_row * r_row

    @pl.when(i == steps - 1)
    def _final():
        n = jnp.float32(steps * _NB)
        s1 = jnp.sum(acc1_ref[...])
        s2 = jnp.sum(acc2_ref[...])
        bm = s1 / n
        bm2 = s2 - n * bm * bm
        cnt = count_ref[0]
        new_count = cnt + n
        delta = bm - mean_ref[0]
        new_mean = mean_ref[0] + delta * n / new_count
        new_m2 = m2_ref[0] + bm2 + delta * delta * cnt * n / new_count
        std = jnp.where(new_count > 1.0, jnp.sqrt(new_m2 / (new_count - 1.0)), 1.0)
        inv = 1.0 / (std + 1e-8)
        r_all = racc_ref[...].reshape(steps, _NB)
        out_ref[...] = (r_all - new_mean) * inv


def kernel(obs, reward_mean, reward_m2, reward_count,
           tW1, tb1, tW2, tb2, tW3, tb3,
           pW1, pb1, pW2, pb2, pW3, pb3):
    batch, obs_dim = obs.shape
    z = jnp.zeros((_H, _H), jnp.float32)
    w1 = jnp.concatenate([tW1, pW1], axis=0)
    w2 = jnp.concatenate(
        [jnp.concatenate([tW2, z], axis=1),
         jnp.concatenate([z, pW2], axis=1)], axis=0)
    w3 = jnp.concatenate([tW3, -pW3], axis=1)
    b1 = jnp.broadcast_to(jnp.concatenate([tb1, pb1])[:, None], (_W, 128))
    b2 = jnp.broadcast_to(jnp.concatenate([tb2, pb2])[:, None], (_W, 128))
    b3 = jnp.broadcast_to((tb3 - pb3)[:, None], (_H, 128))

    xt = obs.T
    steps = batch // _NB

    normalized = pl.pallas_call(
        functools.partial(_body, steps),
        grid=(steps,),
        in_specs=[
            pl.BlockSpec((obs_dim, _NB), lambda i: (0, i)),
            pl.BlockSpec((_W, obs_dim), lambda i: (0, 0)),
            pl.BlockSpec((_W, 128), lambda i: (0, 0)),
            pl.BlockSpec((_W, _W), lambda i: (0, 0)),
            pl.BlockSpec((_W, 128), lambda i: (0, 0)),
            pl.BlockSpec((_H, _W), lambda i: (0, 0)),
            pl.BlockSpec((_H, 128), lambda i: (0, 0)),
            pl.BlockSpec(memory_space=pltpu.SMEM),
            pl.BlockSpec(memory_space=pltpu.SMEM),
            pl.BlockSpec(memory_space=pltpu.SMEM),
        ],
        out_specs=pl.BlockSpec((steps, _NB), lambda i: (0, 0)),
        out_shape=jax.ShapeDtypeStruct((steps, _NB), jnp.float32),
        scratch_shapes=[
            pltpu.VMEM((steps, 1, _NB), jnp.float32),
            pltpu.VMEM((1, _NB), jnp.float32),
            pltpu.VMEM((1, _NB), jnp.float32),
        ],
        compiler_params=pltpu.CompilerParams(
            dimension_semantics=("arbitrary",),
        ),
    )(xt, w1, b1, w2, b2, w3, b3, reward_mean, reward_m2, reward_count)

    return normalized.reshape(batch)
